# Initial kernel scaffold; baseline (speedup 1.0000x reference)
import jax
import jax.numpy as jnp
from jax.experimental import pallas as pl


def kernel(x, edge_index, W_g, b_g, W_l, b_l):
    raise NotImplementedError("write your pallas kernel here")


# speedup vs baseline: 21.4264x; 21.4264x over previous
import functools

import jax
import jax.numpy as jnp
from jax import lax
from jax.experimental import pallas as pl
from jax.experimental.pallas import tpu as pltpu
from jax.experimental.pallas import tpu_sc as plsc

N_NODES = 10000
N_EDGES = 320000
D_IN = 128
D_HID = 128
D_OUT = 64

NC = 2
NS = 16
NW = NC * NS
K = 128
NCHUNK = N_EDGES // K
FULL = NCHUNK // NW
EXTRA = NCHUNK - FULL * NW
N_PAD = 10240
ROWS_PER_TILE = N_PAD // NS

_mesh = lambda: plsc.VectorSubcoreMesh(core_axis_name="c", subcore_axis_name="s")


def _worker_id():
    return lax.axis_index("s") * NC + lax.axis_index("c")


def _zero_rows(buf, nrows, width):
    def fz(i, _):
        for j in range(width // 16):
            buf[i, pl.ds(j * 16, 16)] = jnp.zeros((16,), jnp.float32)
        return 0
    lax.fori_loop(0, nrows, fz, 0, unroll=False)


def _zero_acc_slice(acc, zbuf, base):
    for i in range(5):
        pltpu.sync_copy(zbuf.at[pl.ds(0, 128)], acc.at[pl.ds(base + i * 128, 128)])


def _chunk_count(w):
    return FULL + jnp.where(w < EXTRA, 1, 0)


@functools.partial(
    pl.kernel,
    out_type=jax.ShapeDtypeStruct((NC, 1, N_PAD), jnp.float32),
    mesh=_mesh(),
    scratch_types=[
        pltpu.VMEM((K,), jnp.float32),
        pltpu.VMEM((ROWS_PER_TILE,), jnp.float32),
        pltpu.VMEM((K,), jnp.int32),
        pltpu.VMEM_SHARED((N_PAD,), jnp.float32),
    ],
)
def _deg_kernel(dst_hbm, deg_hbm, ones_v, zbuf, didx, acc):
    w = _worker_id()
    s_ax = lax.axis_index("s")
    c_ax = lax.axis_index("c")

    for j in range(K // 16):
        ones_v[pl.ds(j * 16, 16)] = jnp.ones((16,), jnp.float32)
    for j in range(ROWS_PER_TILE // 16):
        zbuf[pl.ds(j * 16, 16)] = jnp.zeros((16,), jnp.float32)

    base = s_ax * ROWS_PER_TILE
    pltpu.sync_copy(zbuf, acc.at[pl.ds(base, ROWS_PER_TILE)])
    plsc.subcore_barrier()

    def body(j, _):
        c = w + NW * j
        pltpu.sync_copy(dst_hbm.at[c], didx)
        pltpu.sync_copy(ones_v, acc.at[didx], add=True)
        return 0
    lax.fori_loop(0, _chunk_count(w), body, 0, unroll=False)

    plsc.subcore_barrier()
    pltpu.sync_copy(acc.at[pl.ds(base, ROWS_PER_TILE)],
                    deg_hbm.at[c_ax, 0, pl.ds(base, ROWS_PER_TILE)])


@functools.partial(
    pl.kernel,
    out_type=jax.ShapeDtypeStruct((NC, N_PAD, D_HID), jnp.float32),
    mesh=_mesh(),
    scratch_types=[
        pltpu.VMEM((K, D_HID), jnp.float32),
        pltpu.VMEM((K,), jnp.int32),
        pltpu.VMEM((K,), jnp.int32),
        pltpu.VMEM_SHARED((N_PAD, D_HID), jnp.float32),
        pltpu.SemaphoreType.DMA,
    ],
)
def _agg_kernel(xws_hbm, src_hbm, dst_hbm, agg_hbm, rows, sidx, didx, acc, sem):
    w = _worker_id()
    s_ax = lax.axis_index("s")
    c_ax = lax.axis_index("c")

    _zero_rows(rows, K, D_HID)
    base = s_ax * ROWS_PER_TILE
    _zero_acc_slice(acc, rows, base)
    plsc.subcore_barrier()

    def body(j, _):
        c = w + NW * j
        pltpu.sync_copy(src_hbm.at[c], sidx)
        pltpu.sync_copy(dst_hbm.at[c], didx)
        pltpu.async_copy(xws_hbm.at[sidx], rows, sem).wait()
        pltpu.sync_copy(rows, acc.at[didx], add=True)
        return 0
    lax.fori_loop(0, _chunk_count(w), body, 0, unroll=False)

    plsc.subcore_barrier()
    pltpu.sync_copy(acc.at[pl.ds(base, ROWS_PER_TILE)],
                    agg_hbm.at[c_ax, pl.ds(base, ROWS_PER_TILE)])


RB = 1000


def _xws_body(x_ref, w_ref, deg_ref, o_ref):
    deg = deg_ref[0, :, 0:1] + deg_ref[1, :, 0:1] + 1.0
    dinv = lax.rsqrt(deg)
    xw = jnp.dot(x_ref[...], w_ref[...], preferred_element_type=jnp.float32)
    o_ref[...] = xw * dinv


def _xws_call(x, W_g, deg_parts):
    return pl.pallas_call(
        _xws_body,
        grid=(N_NODES // RB,),
        in_specs=[
            pl.BlockSpec((RB, D_IN), lambda i: (i, 0)),
            pl.BlockSpec((D_IN, D_HID), lambda i: (0, 0)),
            pl.BlockSpec((NC, RB, 1), lambda i: (0, i, 0)),
        ],
        out_specs=pl.BlockSpec((RB, D_HID), lambda i: (i, 0)),
        out_shape=jax.ShapeDtypeStruct((N_NODES, D_HID), jnp.float32),
    )(x, W_g, deg_parts)


def _head_body(agg_ref, xws_ref, deg_ref, bg_ref, wl_ref, bl_ref, o_ref):
    deg = deg_ref[0, :, 0:1] + deg_ref[1, :, 0:1] + 1.0
    dinv = lax.rsqrt(deg)
    pre = (agg_ref[0] + agg_ref[1] + xws_ref[...]) * dinv + bg_ref[...]
    h = jnp.maximum(pre, 0.0)
    o_ref[...] = jnp.dot(h, wl_ref[...], preferred_element_type=jnp.float32) + bl_ref[...]


def _head_call(agg_parts, xws, deg_parts, b_g, W_l, b_l):
    return pl.pallas_call(
        _head_body,
        grid=(N_NODES // RB,),
        in_specs=[
            pl.BlockSpec((NC, RB, D_HID), lambda i: (0, i, 0)),
            pl.BlockSpec((RB, D_HID), lambda i: (i, 0)),
            pl.BlockSpec((NC, RB, 1), lambda i: (0, i, 0)),
            pl.BlockSpec((1, D_HID), lambda i: (0, 0)),
            pl.BlockSpec((D_HID, D_OUT), lambda i: (0, 0)),
            pl.BlockSpec((1, D_OUT), lambda i: (0, 0)),
        ],
        out_specs=pl.BlockSpec((RB, D_OUT), lambda i: (i, 0)),
        out_shape=jax.ShapeDtypeStruct((N_NODES, D_OUT), jnp.float32),
    )(agg_parts, xws, deg_parts, b_g, W_l, b_l)


def kernel(x, edge_index, W_g, b_g, W_l, b_l):
    src = edge_index[0].astype(jnp.int32).reshape(NCHUNK, K)
    dst = edge_index[1].astype(jnp.int32).reshape(NCHUNK, K)
    deg_parts = _deg_kernel(dst).reshape(NC, N_PAD, 1)
    xws = _xws_call(x, W_g, deg_parts)
    agg_parts = _agg_kernel(xws, src, dst)
    return _head_call(agg_parts, xws, deg_parts,
                      b_g.reshape(1, D_HID), W_l, b_l.reshape(1, D_OUT))
